# baseline (speedup 1.0000x reference)
import jax
import jax.numpy as jnp
from jax import lax
from jax.experimental import pallas as pl
from jax.experimental.pallas import tpu as pltpu
from jax.experimental.pallas import tpu_sc as plsc

MAX_LEN = 1024
DIM = 256
T = 1024
EPAD = 2048
BB = 8
L = 16


def _sc_rel_sum(e_hbm, o_hbm, ebuf, obuf, sem):
    c = lax.axis_index("c")
    s = lax.axis_index("s")
    pltpu.async_copy(e_hbm.at[:, pl.ds(s * L, L)], ebuf, sem).wait()

    @pl.loop(1, EPAD - 1)
    def _(i):
        ebuf[i] = ebuf[i] + ebuf[i - 1]

    half = T // 2
    base = c * half

    @pl.loop(0, half)
    def _(k):
        gi = base + k
        t1 = 2 * MAX_LEN - 2 - gi
        t2 = jnp.where(gi == MAX_LEN - 1, EPAD - 1, MAX_LEN - 2 - gi)
        obuf[k] = ebuf[t1] - ebuf[t2]

    pltpu.async_copy(obuf, o_hbm.at[pl.ds(base, half), pl.ds(s * L, L)],
                     sem).wait()


def _tc_add(rs_ref, x_ref, o_ref):
    o_ref[:] = x_ref[:] + rs_ref[:][None]


def kernel(x, rel_embedding):
    b, t, d = x.shape
    e_pad = jnp.concatenate(
        [rel_embedding, jnp.zeros((1, d), rel_embedding.dtype)], axis=0)

    rel_sum = pl.kernel(
        _sc_rel_sum,
        out_type=jax.ShapeDtypeStruct((t, d), jnp.float32),
        mesh=plsc.VectorSubcoreMesh(core_axis_name="c", subcore_axis_name="s"),
        scratch_types=[
            pltpu.VMEM((EPAD, L), jnp.float32),
            pltpu.VMEM((T // 2, L), jnp.float32),
            pltpu.SemaphoreType.DMA,
        ],
        compiler_params=pltpu.CompilerParams(use_tc_tiling_on_sc=False),
    )(e_pad)

    return pl.pallas_call(
        _tc_add,
        grid=(b // BB,),
        in_specs=[
            pl.BlockSpec((t, d), lambda i: (0, 0)),
            pl.BlockSpec((BB, t, d), lambda i: (i, 0, 0)),
        ],
        out_specs=pl.BlockSpec((BB, t, d), lambda i: (i, 0, 0)),
        out_shape=jax.ShapeDtypeStruct((b, t, d), x.dtype),
    )(rel_sum, x)


# speedup vs baseline: 3.3165x; 3.3165x over previous
import jax
import jax.numpy as jnp
from jax.experimental import pallas as pl
from jax.experimental.pallas import tpu as pltpu

MAX_LEN = 1024
DIM = 256
T = 1024
EPAD = 2048
CH = 2
NCHUNK = 16 // CH
NBUF = 4


def _body(e_ref, x_hbm, o_hbm, rs_ref, inbuf, outbuf, in_sems, out_sems):
    def in_copy(i):
        return pltpu.make_async_copy(
            x_hbm.at[pl.ds(i * CH, CH)], inbuf.at[i % NBUF],
            in_sems.at[i % NBUF])

    def out_copy(i):
        return pltpu.make_async_copy(
            outbuf.at[i % NBUF], o_hbm.at[pl.ds(i * CH, CH)],
            out_sems.at[i % NBUF])

    for i in range(NBUF):
        in_copy(i).start()

    ii = jax.lax.broadcasted_iota(jnp.int32, (T, EPAD), 0)
    kk = jax.lax.broadcasted_iota(jnp.int32, (T, EPAD), 1)
    ss = ii + kk
    w = jnp.where((ss >= MAX_LEN - 1) & (ss <= 2 * MAX_LEN - 2), 1.0, 0.0)
    rs_ref[:] = jnp.dot(w.astype(jnp.float32), e_ref[:],
                        preferred_element_type=jnp.float32)

    for i in range(NCHUNK):
        s = i % NBUF
        in_copy(i).wait()
        if i >= NBUF:
            out_copy(i - NBUF).wait()
        outbuf[s] = inbuf[s] + rs_ref[:][None]
        out_copy(i).start()
        if i + NBUF < NCHUNK:
            in_copy(i + NBUF).start()

    for i in range(NCHUNK - NBUF, NCHUNK):
        out_copy(i).wait()


def kernel(x, rel_embedding):
    b, t, d = x.shape
    e_pad = jnp.concatenate(
        [rel_embedding, jnp.zeros((1, d), rel_embedding.dtype)], axis=0)

    return pl.pallas_call(
        _body,
        in_specs=[
            pl.BlockSpec((EPAD, d), lambda: (0, 0)),
            pl.BlockSpec(memory_space=pl.ANY),
        ],
        out_specs=pl.BlockSpec(memory_space=pl.ANY),
        out_shape=jax.ShapeDtypeStruct((b, t, d), x.dtype),
        scratch_shapes=[
            pltpu.VMEM((t, d), jnp.float32),
            pltpu.VMEM((NBUF, CH, t, d), jnp.float32),
            pltpu.VMEM((NBUF, CH, t, d), jnp.float32),
            pltpu.SemaphoreType.DMA((NBUF,)),
            pltpu.SemaphoreType.DMA((NBUF,)),
        ],
    )(e_pad, x)
